# baseline (speedup 1.0000x reference)
import functools

import jax
import jax.numpy as jnp
from jax import lax
from jax.experimental import pallas as pl
from jax.experimental.pallas import tpu as pltpu
from jax.experimental.pallas import tpu_sc as plsc

NREL = 25
NND = 10000
NEDGE = 320000
NFEAT = 128
FD = 16
NC, NS = 2, 16
GB = 400
CHUNK = 2000
NB = CHUNK // GB
EPT = NEDGE // NS
NCHUNK = EPT // CHUNK
GPT = EPT // GB
RPT = NND // NS
RPC = 13
BM = 2000


def _spmm_pipeline(sup_hbm, rows_hbm, cols_hbm, vals_hbm,
                   cols_v, rows_v, vals_v, gath_v, acc_sh,
                   lsem, gsem, ssem, s, i):
    def _fire_loads(j, b):
        ci = s * NCHUNK + j
        pltpu.async_copy(cols_hbm.at[i, pl.ds(ci * NB, NB), :], cols_v.at[b],
                         lsem)
        pltpu.async_copy(rows_hbm.at[i, pl.ds(ci * NB, NB), :], rows_v.at[b],
                         lsem)
        pltpu.async_copy(vals_hbm.at[i, ci, :], vals_v.at[b], lsem)

    def _drain_loads(j, b):
        ci = s * NCHUNK + j
        pltpu.make_async_copy(cols_hbm.at[i, pl.ds(ci * NB, NB), :],
                              cols_v.at[b], lsem).wait()
        pltpu.make_async_copy(rows_hbm.at[i, pl.ds(ci * NB, NB), :],
                              rows_v.at[b], lsem).wait()
        pltpu.make_async_copy(vals_hbm.at[i, ci, :], vals_v.at[b],
                              lsem).wait()

    def _addoff(b):
        for g in range(NB):
            @plsc.parallel_loop(0, GB // 16)
            def _body(q):
                sl = pl.ds(q * 16, 16)
                cols_v[b, g, sl] = cols_v[b, g, sl] * NREL + i

    def _fire_gathers(b):
        for g in range(NB):
            pltpu.async_copy(sup_hbm.at[cols_v.at[b, g]],
                             gath_v.at[b].at[pl.ds(g * GB, GB)], gsem)

    def _drain_gathers(b):
        for g in range(NB):
            pltpu.make_async_copy(sup_hbm.at[cols_v.at[b, g]],
                                  gath_v.at[b].at[pl.ds(g * GB, GB)],
                                  gsem).wait()

    def _scale(b):
        @plsc.parallel_loop(0, CHUNK // 16)
        def _body(q):
            base = q * 16
            vv = vals_v[b, pl.ds(base, 16)]
            for u in range(16):
                e = base + u
                gath_v[b, e, :] = gath_v[b, e, :] * vv[u]

    def _fire_scatters(b):
        for g in range(NB):
            pltpu.async_copy(gath_v.at[b].at[pl.ds(g * GB, GB)],
                             acc_sh.at[rows_v.at[b, g]], ssem, add=True)

    def _drain_scatters(b):
        for g in range(NB):
            pltpu.make_async_copy(gath_v.at[b].at[pl.ds(g * GB, GB)],
                                  acc_sh.at[rows_v.at[b, g]], ssem).wait()

    _fire_loads(0, 0)
    _drain_loads(0, 0)
    _addoff(0)
    _fire_gathers(0)
    for j in range(NCHUNK):
        b = j % 2
        if j + 1 < NCHUNK:
            if j >= 1:
                _drain_scatters(1 - b)
            _fire_loads(j + 1, 1 - b)
        _drain_gathers(b)
        if j + 1 < NCHUNK:
            _drain_loads(j + 1, 1 - b)
            _addoff(1 - b)
            _fire_gathers(1 - b)
        _scale(b)
        _fire_scatters(b)
    _drain_scatters(0)
    _drain_scatters(1)


def _zero_buf(zero_v):
    @plsc.parallel_loop(0, RPT)
    def _zb(r):
        zero_v[r, :] = jnp.zeros((FD,), jnp.float32)


def _spmm1_body(sup_hbm, rows_hbm, cols_hbm, vals_hbm, b_hbm, h_hbm,
                cols_v, rows_v, vals_v, gath_v, zero_v, tmp_v, bv_v,
                acc_sh, lsem, gsem, ssem):
    c = lax.axis_index("c")
    s = lax.axis_index("s")
    rbase = s * RPT
    _zero_buf(zero_v)
    pltpu.sync_copy(b_hbm, bv_v)

    def _rel(k, carry):
        i = c * RPC + k

        @pl.when(i < NREL)
        def _():
            pltpu.sync_copy(zero_v, acc_sh.at[pl.ds(rbase, RPT)])
            plsc.subcore_barrier()
            _spmm_pipeline(sup_hbm, rows_hbm, cols_hbm, vals_hbm,
                           cols_v, rows_v, vals_v, gath_v, acc_sh,
                           lsem, gsem, ssem, s, i)
            plsc.subcore_barrier()
            pltpu.sync_copy(acc_sh.at[pl.ds(rbase, RPT)], tmp_v)
            bv = bv_v[i, :]

            @plsc.parallel_loop(0, RPT)
            def _br(r):
                tmp_v[r, :] = jnp.maximum(tmp_v[r, :] + bv, 0.0)
            pltpu.sync_copy(tmp_v, h_hbm.at[pl.ds(rbase, RPT), i, :])
        return carry
    lax.fori_loop(0, RPC, _rel, 0)


def _spmm2_body(sup_hbm, rows_hbm, cols_hbm, vals_hbm, b_hbm, pmax_hbm,
                cols_v, rows_v, vals_v, gath_v, zero_v, tmp_v, bv_v,
                runmax_v, acc_sh, lsem, gsem, ssem):
    c = lax.axis_index("c")
    s = lax.axis_index("s")
    rbase = s * RPT
    _zero_buf(zero_v)
    pltpu.sync_copy(b_hbm, bv_v)

    def _rel(k, carry):
        i = c * RPC + k

        @pl.when(i < NREL)
        def _():
            pltpu.sync_copy(zero_v, acc_sh.at[pl.ds(rbase, RPT)])
            plsc.subcore_barrier()
            _spmm_pipeline(sup_hbm, rows_hbm, cols_hbm, vals_hbm,
                           cols_v, rows_v, vals_v, gath_v, acc_sh,
                           lsem, gsem, ssem, s, i)
            plsc.subcore_barrier()
            pltpu.sync_copy(acc_sh.at[pl.ds(rbase, RPT)], tmp_v)
            bv = bv_v[i, :]

            @pl.when(k == 0)
            def _():
                @plsc.parallel_loop(0, RPT)
                def _init(r):
                    runmax_v[r, :] = tmp_v[r, :] + bv

            @pl.when(k > 0)
            def _():
                @plsc.parallel_loop(0, RPT)
                def _merge(r):
                    runmax_v[r, :] = jnp.maximum(runmax_v[r, :],
                                                 tmp_v[r, :] + bv)
        return carry
    lax.fori_loop(0, RPC, _rel, 0)
    pltpu.sync_copy(runmax_v, pmax_hbm.at[c, pl.ds(rbase, RPT), :])


_SPMM_SCRATCH = (
    pltpu.VMEM((2, NB, GB), jnp.int32),
    pltpu.VMEM((2, NB, GB), jnp.int32),
    pltpu.VMEM((2, CHUNK), jnp.float32),
    pltpu.VMEM((2, CHUNK, FD), jnp.float32),
    pltpu.VMEM((RPT, FD), jnp.float32),
    pltpu.VMEM((RPT, FD), jnp.float32),
    pltpu.VMEM((NREL, FD), jnp.float32),
)


def _sc_mesh():
    return plsc.VectorSubcoreMesh(core_axis_name="c", subcore_axis_name="s",
                                  num_cores=NC, num_subcores=NS)


@functools.lru_cache(maxsize=None)
def _make_spmm1():
    return pl.kernel(
        _spmm1_body,
        out_type=jax.ShapeDtypeStruct((NND, NREL, FD), jnp.float32),
        mesh=_sc_mesh(),
        compiler_params=pltpu.CompilerParams(use_tc_tiling_on_sc=False),
        scratch_types=[
            *_SPMM_SCRATCH,
            pltpu.VMEM_SHARED((NND, FD), jnp.float32),
            pltpu.SemaphoreType.DMA,
            pltpu.SemaphoreType.DMA,
            pltpu.SemaphoreType.DMA,
        ],
    )


@functools.lru_cache(maxsize=None)
def _make_spmm2():
    return pl.kernel(
        _spmm2_body,
        out_type=jax.ShapeDtypeStruct((NC, NND, FD), jnp.float32),
        mesh=_sc_mesh(),
        compiler_params=pltpu.CompilerParams(use_tc_tiling_on_sc=False),
        scratch_types=[
            *_SPMM_SCRATCH,
            pltpu.VMEM((RPT, FD), jnp.float32),
            pltpu.VMEM_SHARED((NND, FD), jnp.float32),
            pltpu.SemaphoreType.DMA,
            pltpu.SemaphoreType.DMA,
            pltpu.SemaphoreType.DMA,
        ],
    )


def _mm_body(x_ref, w_ref, o_ref):
    o_ref[...] = jnp.dot(x_ref[...], w_ref[...],
                         preferred_element_type=jnp.float32,
                         precision=lax.Precision.HIGHEST)


def _maxfin_body(p_ref, o_ref):
    o_ref[...] = jnp.maximum(jnp.maximum(p_ref[0], p_ref[1]), 0.0)


def _mm(x, w):
    m, k = x.shape
    n = w.shape[1]
    return pl.pallas_call(
        _mm_body,
        grid=(m // BM,),
        in_specs=[pl.BlockSpec((BM, k), lambda mm_: (mm_, 0)),
                  pl.BlockSpec((k, n), lambda mm_: (0, 0))],
        out_specs=pl.BlockSpec((BM, n), lambda mm_: (mm_, 0)),
        out_shape=jax.ShapeDtypeStruct((m, n), jnp.float32),
    )(x, w)


def _maxfin(p):
    return pl.pallas_call(
        _maxfin_body,
        grid=(NND // BM,),
        in_specs=[pl.BlockSpec((NC, BM, FD), lambda m: (0, m, 0))],
        out_specs=pl.BlockSpec((BM, FD), lambda m: (m, 0)),
        out_shape=jax.ShapeDtypeStruct((NND, FD), jnp.float32),
    )(p)


def kernel(x, adj_indices, adj_values, W1, b1, W2, b2):
    adj_indices = adj_indices.astype(jnp.int32)
    adj_rows = adj_indices[:, 0, :].reshape(NREL, NEDGE // GB, GB)
    adj_cols = adj_indices[:, 1, :].reshape(NREL, NEDGE // GB, GB)
    vals = adj_values.astype(jnp.float32).reshape(NREL, NEDGE // CHUNK, CHUNK)

    W1f = W1.transpose(1, 0, 2).reshape(NFEAT, NREL * FD)
    W2f = W2.transpose(1, 0, 2).reshape(NREL * FD, NREL * FD)

    s1f = _mm(x, W1f)
    h3 = _make_spmm1()(s1f.reshape(NND * NREL, FD), adj_rows, adj_cols,
                       vals, b1)
    s2f = _mm(h3.reshape(NND, NREL * FD), W2f)
    pmax = _make_spmm2()(s2f.reshape(NND * NREL, FD), adj_rows, adj_cols,
                         vals, b2)
    return _maxfin(pmax)


# speedup vs baseline: 1.0624x; 1.0624x over previous
import functools

import jax
import jax.numpy as jnp
from jax import lax
from jax.experimental import pallas as pl
from jax.experimental.pallas import tpu as pltpu
from jax.experimental.pallas import tpu_sc as plsc

NREL = 25
NND = 10000
NEDGE = 320000
NFEAT = 128
FD = 16
NC, NS = 2, 16
GB = 80
CHUNK = 2000
NB = CHUNK // GB
EPT = NEDGE // NS
NCHUNK = EPT // CHUNK
GPT = EPT // GB
RPT = NND // NS
RPC = 13
BM = 2000


def _spmm_pipeline(sup_hbm, rows_hbm, cols_hbm, vals_hbm,
                   cols_v, rows_v, vals_v, gath_v, acc_sh,
                   lsem, gsem, ssem, s, i):
    def _fire_loads(j, b):
        ci = s * NCHUNK + j
        pltpu.async_copy(cols_hbm.at[i, pl.ds(ci * NB, NB), :], cols_v.at[b],
                         lsem)
        pltpu.async_copy(rows_hbm.at[i, pl.ds(ci * NB, NB), :], rows_v.at[b],
                         lsem)
        pltpu.async_copy(vals_hbm.at[i, ci, :], vals_v.at[b], lsem)

    def _drain_loads(j, b):
        ci = s * NCHUNK + j
        pltpu.make_async_copy(cols_hbm.at[i, pl.ds(ci * NB, NB), :],
                              cols_v.at[b], lsem).wait()
        pltpu.make_async_copy(rows_hbm.at[i, pl.ds(ci * NB, NB), :],
                              rows_v.at[b], lsem).wait()
        pltpu.make_async_copy(vals_hbm.at[i, ci, :], vals_v.at[b],
                              lsem).wait()

    def _addoff(b):
        @plsc.parallel_loop(0, NB)
        def _body(g):
            for t in range(GB // 16):
                sl = pl.ds(t * 16, 16)
                cols_v[b, g, sl] = cols_v[b, g, sl] * NREL + i

    def _fire_gathers(b):
        for g in range(NB):
            pltpu.async_copy(sup_hbm.at[cols_v.at[b, g]],
                             gath_v.at[b].at[pl.ds(g * GB, GB)], gsem)

    def _drain_gathers(b):
        for g in range(NB):
            pltpu.make_async_copy(sup_hbm.at[cols_v.at[b, g]],
                                  gath_v.at[b].at[pl.ds(g * GB, GB)],
                                  gsem).wait()

    def _scale(b):
        @plsc.parallel_loop(0, CHUNK // 16)
        def _body(q):
            base = q * 16
            vv = vals_v[b, pl.ds(base, 16)]
            for u in range(16):
                e = base + u
                gath_v[b, e, :] = gath_v[b, e, :] * vv[u]

    def _fire_scatters(b):
        for g in range(NB):
            pltpu.async_copy(gath_v.at[b].at[pl.ds(g * GB, GB)],
                             acc_sh.at[rows_v.at[b, g]], ssem, add=True)

    def _drain_scatters(b):
        for g in range(NB):
            pltpu.make_async_copy(gath_v.at[b].at[pl.ds(g * GB, GB)],
                                  acc_sh.at[rows_v.at[b, g]], ssem).wait()

    _fire_loads(0, 0)
    _drain_loads(0, 0)
    _addoff(0)
    _fire_gathers(0)
    for j in range(NCHUNK):
        b = j % 2
        if j + 1 < NCHUNK:
            if j >= 1:
                _drain_scatters(1 - b)
            _fire_loads(j + 1, 1 - b)
        _drain_gathers(b)
        if j + 1 < NCHUNK:
            _drain_loads(j + 1, 1 - b)
            _addoff(1 - b)
            _fire_gathers(1 - b)
        _scale(b)
        _fire_scatters(b)
    _drain_scatters(0)
    _drain_scatters(1)


def _zero_buf(zero_v):
    @plsc.parallel_loop(0, RPT)
    def _zb(r):
        zero_v[r, :] = jnp.zeros((FD,), jnp.float32)


def _spmm1_body(sup_hbm, rows_hbm, cols_hbm, vals_hbm, b_hbm, h_hbm,
                cols_v, rows_v, vals_v, gath_v, zero_v, tmp_v, bv_v,
                acc_sh, lsem, gsem, ssem):
    c = lax.axis_index("c")
    s = lax.axis_index("s")
    rbase = s * RPT
    _zero_buf(zero_v)
    pltpu.sync_copy(b_hbm, bv_v)

    def _rel(k, carry):
        i = c * RPC + k

        @pl.when(i < NREL)
        def _():
            pltpu.sync_copy(zero_v, acc_sh.at[pl.ds(rbase, RPT)])
            plsc.subcore_barrier()
            _spmm_pipeline(sup_hbm, rows_hbm, cols_hbm, vals_hbm,
                           cols_v, rows_v, vals_v, gath_v, acc_sh,
                           lsem, gsem, ssem, s, i)
            plsc.subcore_barrier()
            pltpu.sync_copy(acc_sh.at[pl.ds(rbase, RPT)], tmp_v)
            bv = bv_v[i, :]

            @plsc.parallel_loop(0, RPT)
            def _br(r):
                tmp_v[r, :] = jnp.maximum(tmp_v[r, :] + bv, 0.0)
            pltpu.sync_copy(tmp_v, h_hbm.at[pl.ds(rbase, RPT), i, :])
        return carry
    lax.fori_loop(0, RPC, _rel, 0)


def _spmm2_body(sup_hbm, rows_hbm, cols_hbm, vals_hbm, b_hbm, pmax_hbm,
                cols_v, rows_v, vals_v, gath_v, zero_v, tmp_v, bv_v,
                runmax_v, acc_sh, lsem, gsem, ssem):
    c = lax.axis_index("c")
    s = lax.axis_index("s")
    rbase = s * RPT
    _zero_buf(zero_v)
    pltpu.sync_copy(b_hbm, bv_v)

    def _rel(k, carry):
        i = c * RPC + k

        @pl.when(i < NREL)
        def _():
            pltpu.sync_copy(zero_v, acc_sh.at[pl.ds(rbase, RPT)])
            plsc.subcore_barrier()
            _spmm_pipeline(sup_hbm, rows_hbm, cols_hbm, vals_hbm,
                           cols_v, rows_v, vals_v, gath_v, acc_sh,
                           lsem, gsem, ssem, s, i)
            plsc.subcore_barrier()
            pltpu.sync_copy(acc_sh.at[pl.ds(rbase, RPT)], tmp_v)
            bv = bv_v[i, :]

            @pl.when(k == 0)
            def _():
                @plsc.parallel_loop(0, RPT)
                def _init(r):
                    runmax_v[r, :] = tmp_v[r, :] + bv

            @pl.when(k > 0)
            def _():
                @plsc.parallel_loop(0, RPT)
                def _merge(r):
                    runmax_v[r, :] = jnp.maximum(runmax_v[r, :],
                                                 tmp_v[r, :] + bv)
        return carry
    lax.fori_loop(0, RPC, _rel, 0)
    pltpu.sync_copy(runmax_v, pmax_hbm.at[c, pl.ds(rbase, RPT), :])


_SPMM_SCRATCH = (
    pltpu.VMEM((2, NB, GB), jnp.int32),
    pltpu.VMEM((2, NB, GB), jnp.int32),
    pltpu.VMEM((2, CHUNK), jnp.float32),
    pltpu.VMEM((2, CHUNK, FD), jnp.float32),
    pltpu.VMEM((RPT, FD), jnp.float32),
    pltpu.VMEM((RPT, FD), jnp.float32),
    pltpu.VMEM((NREL, FD), jnp.float32),
)


def _sc_mesh():
    return plsc.VectorSubcoreMesh(core_axis_name="c", subcore_axis_name="s",
                                  num_cores=NC, num_subcores=NS)


@functools.lru_cache(maxsize=None)
def _make_spmm1():
    return pl.kernel(
        _spmm1_body,
        out_type=jax.ShapeDtypeStruct((NND, NREL, FD), jnp.float32),
        mesh=_sc_mesh(),
        compiler_params=pltpu.CompilerParams(use_tc_tiling_on_sc=False),
        scratch_types=[
            *_SPMM_SCRATCH,
            pltpu.VMEM_SHARED((NND, FD), jnp.float32),
            pltpu.SemaphoreType.DMA,
            pltpu.SemaphoreType.DMA,
            pltpu.SemaphoreType.DMA,
        ],
    )


@functools.lru_cache(maxsize=None)
def _make_spmm2():
    return pl.kernel(
        _spmm2_body,
        out_type=jax.ShapeDtypeStruct((NC, NND, FD), jnp.float32),
        mesh=_sc_mesh(),
        compiler_params=pltpu.CompilerParams(use_tc_tiling_on_sc=False),
        scratch_types=[
            *_SPMM_SCRATCH,
            pltpu.VMEM((RPT, FD), jnp.float32),
            pltpu.VMEM_SHARED((NND, FD), jnp.float32),
            pltpu.SemaphoreType.DMA,
            pltpu.SemaphoreType.DMA,
            pltpu.SemaphoreType.DMA,
        ],
    )


def _mm_body(x_ref, w_ref, o_ref):
    o_ref[...] = jnp.dot(x_ref[...], w_ref[...],
                         preferred_element_type=jnp.float32,
                         precision=lax.Precision.HIGHEST)


def _maxfin_body(p_ref, o_ref):
    o_ref[...] = jnp.maximum(jnp.maximum(p_ref[0], p_ref[1]), 0.0)


def _mm(x, w):
    m, k = x.shape
    n = w.shape[1]
    return pl.pallas_call(
        _mm_body,
        grid=(m // BM,),
        in_specs=[pl.BlockSpec((BM, k), lambda mm_: (mm_, 0)),
                  pl.BlockSpec((k, n), lambda mm_: (0, 0))],
        out_specs=pl.BlockSpec((BM, n), lambda mm_: (mm_, 0)),
        out_shape=jax.ShapeDtypeStruct((m, n), jnp.float32),
    )(x, w)


def _maxfin(p):
    return pl.pallas_call(
        _maxfin_body,
        grid=(NND // BM,),
        in_specs=[pl.BlockSpec((NC, BM, FD), lambda m: (0, m, 0))],
        out_specs=pl.BlockSpec((BM, FD), lambda m: (m, 0)),
        out_shape=jax.ShapeDtypeStruct((NND, FD), jnp.float32),
    )(p)


def kernel(x, adj_indices, adj_values, W1, b1, W2, b2):
    adj_indices = adj_indices.astype(jnp.int32)
    adj_rows = adj_indices[:, 0, :].reshape(NREL, NEDGE // GB, GB)
    adj_cols = adj_indices[:, 1, :].reshape(NREL, NEDGE // GB, GB)
    vals = adj_values.astype(jnp.float32).reshape(NREL, NEDGE // CHUNK, CHUNK)

    W1f = W1.transpose(1, 0, 2).reshape(NFEAT, NREL * FD)
    W2f = W2.transpose(1, 0, 2).reshape(NREL * FD, NREL * FD)

    s1f = _mm(x, W1f)
    h3 = _make_spmm1()(s1f.reshape(NND * NREL, FD), adj_rows, adj_cols,
                       vals, b1)
    s2f = _mm(h3.reshape(NND, NREL * FD), W2f)
    pmax = _make_spmm2()(s2f.reshape(NND * NREL, FD), adj_rows, adj_cols,
                         vals, b2)
    return _maxfin(pmax)
